# baseline (speedup 1.0000x reference)
import math

import jax
import jax.numpy as jnp
import numpy as np
from jax.experimental import pallas as pl
from jax.experimental.pallas import tpu as pltpu

B = 1024
T = 20
P = 50
D_DATA = 2
D_EMB = 16
D_LAT = 16
D_HID = 32
SOFTPLUS_BIAS = 0.5413248538970947

BB = 256
NB = B // BB
R = BB * P
HALF_LOG_2PI = 0.5 * math.log(2.0 * math.pi)
NEG_LOG_P = -math.log(float(P))
TINY = float(np.finfo(np.float32).tiny)
NORM_LO = float(np.nextafter(np.float32(-1.0), np.float32(0.0)))
NORM_SCALE = float(np.float32(1.0) - np.float32(NORM_LO))
SQRT2 = float(np.float32(np.sqrt(2.0)))


def _np_threefry2x32(k0, k1, c0, c1):
    def rotl(x, r):
        return np.uint32((int(x) << r | int(x) >> (32 - r)) & 0xFFFFFFFF)
    ks0, ks1 = np.uint32(k0), np.uint32(k1)
    ks2 = np.uint32(ks0 ^ ks1 ^ np.uint32(0x1BD11BDA))
    x0 = np.uint32((int(c0) + int(ks0)) & 0xFFFFFFFF)
    x1 = np.uint32((int(c1) + int(ks1)) & 0xFFFFFFFF)
    rots = [(13, 15, 26, 6), (17, 29, 16, 24)]
    adds = [(ks1, ks2, 1), (ks2, ks0, 2), (ks0, ks1, 3), (ks1, ks2, 4), (ks2, ks0, 5)]
    for i, (a0, a1, inc) in enumerate(adds):
        for r in rots[i % 2]:
            x0 = np.uint32((int(x0) + int(x1)) & 0xFFFFFFFF)
            x1 = rotl(x1, r)
            x1 = np.uint32(x1 ^ x0)
        x0 = np.uint32((int(x0) + int(a0)) & 0xFFFFFFFF)
        x1 = np.uint32((int(x1) + int(a1) + inc) & 0xFFFFFFFF)
    return x0, x1


def _fold_keys():
    eps_keys, gum_keys = [], []
    for t in range(T):
        for lst, d in ((eps_keys, 2 * t), (gum_keys, 2 * t + 1)):
            f0, f1 = _np_threefry2x32(np.uint32(0), np.uint32(42),
                                      np.uint32(0), np.uint32(d))
            lst.append((int(f0), int(f1), int(f0 ^ f1 ^ 0x1BD11BDA)))
    return eps_keys, gum_keys


_EPS_KEYS, _GUM_KEYS = _fold_keys()

_ERFINV_LT = (2.81022636e-08, 3.43273939e-07, -3.5233877e-06, -4.39150654e-06,
              0.00021858087, -0.00125372503, -0.00417768164, 0.246640727,
              1.50140941)
_ERFINV_GE = (-0.000200214257, 0.000100950558, 0.00134934322, -0.00367342844,
              0.00573950773, -0.0076224613, 0.00943887047, 1.00167406,
              2.83297682)


def _sel_key(t, keys):
    outs = []
    for j in range(3):
        acc = jnp.uint32(keys[0][j])
        for i in range(1, T):
            acc = jax.lax.select(t == i, jnp.uint32(keys[i][j]), acc)
        outs.append(acc)
    return outs


def _threefry_bits(pos_plus_k1, k0, k1, ks2):
    u32 = jnp.uint32
    x1 = pos_plus_k1

    def quarter(x0, x1, rots):
        for r in rots:
            x0 = x0 + x1
            x1 = (x1 << u32(r)) | (x1 >> u32(32 - r))
            x1 = x1 ^ x0
        return x0, x1

    ra, rb = (13, 15, 26, 6), (17, 29, 16, 24)
    x0 = x1 + k0
    x1 = (x1 << u32(ra[0])) | (x1 >> u32(32 - ra[0]))
    x1 = x1 ^ x0
    for r in ra[1:]:
        x0 = x0 + x1
        x1 = (x1 << u32(r)) | (x1 >> u32(32 - r))
        x1 = x1 ^ x0
    x0 = x0 + k1
    x1 = x1 + ks2 + u32(1)
    x0, x1 = quarter(x0, x1, rb)
    x0 = x0 + ks2
    x1 = x1 + k0 + u32(2)
    x0, x1 = quarter(x0, x1, ra)
    x0 = x0 + k0
    x1 = x1 + k1 + u32(3)
    x0, x1 = quarter(x0, x1, rb)
    x0 = x0 + k1
    x1 = x1 + ks2 + u32(4)
    x0, x1 = quarter(x0, x1, ra)
    x0 = x0 + ks2
    x1 = x1 + k0 + u32(5)
    return x0 ^ x1


def _bits_to_unit_float(bits):
    fb = (bits >> jnp.uint32(9)) | jnp.uint32(0x3F800000)
    return jax.lax.bitcast_convert_type(fb, jnp.float32) - 1.0


def _erfinv(x):
    w = -jnp.log1p(-x * x)
    wl = w - 2.5
    p1 = jnp.float32(_ERFINV_LT[0])
    for c in _ERFINV_LT[1:]:
        p1 = p1 * wl + jnp.float32(c)
    ws = jnp.sqrt(w) - 3.0
    p2 = jnp.float32(_ERFINV_GE[0])
    for c in _ERFINV_GE[1:]:
        p2 = p2 * ws + jnp.float32(c)
    return jnp.where(w < 5.0, p1, p2) * x


def _softplus(v):
    return jnp.logaddexp(v, 0.0)


def _normal_logprob(x, mu, sig):
    return -0.5 * ((x - mu) / sig) ** 2 - jnp.log(sig) - HALF_LOG_2PI


def _fivo_step(xt_ref, yt_ref,
               wi_ref, wh_ref, bi_ref, bh_ref,
               enc_ref, encb_ref, pri_ref, prib_ref,
               dec_ref, decb_ref, emb_ref, embb_ref,
               out_ref, z_s, h_s, lw_s, acc_s, pose_s, posg_s):
    ib = pl.program_id(0)
    t = pl.program_id(1)
    f32 = jnp.float32

    @pl.when(t == 0)
    def _init():
        z_s[...] = jnp.zeros_like(z_s)
        h_s[...] = jnp.zeros_like(h_s)
        lw_s[...] = jnp.zeros_like(lw_s)
        acc_s[...] = jnp.zeros_like(acc_s)

    @pl.when((t == 0) & (ib == 0))
    def _init_pos():
        i32 = jnp.int32
        d_i = jax.lax.broadcasted_iota(i32, (D_LAT, P, BB), 0)
        p_i = jax.lax.broadcasted_iota(i32, (D_LAT, P, BB), 1)
        b_i = jax.lax.broadcasted_iota(i32, (D_LAT, P, BB), 2)
        pose_s[...] = (b_i * (P * D_LAT) + p_i * D_LAT + d_i).astype(jnp.uint32)
        k_i = jax.lax.broadcasted_iota(i32, (P, P, BB), 0)
        pn_i = jax.lax.broadcasted_iota(i32, (P, P, BB), 1)
        gb_i = jax.lax.broadcasted_iota(i32, (P, P, BB), 2)
        posg_s[...] = (pn_i * (B * P) + gb_i * P + k_i).astype(jnp.uint32)

    xt = xt_ref[0]
    yt = yt_ref[0]
    zT = z_s[...]
    hT = h_s[...]

    ppT = jnp.dot(pri_ref[...], zT, preferred_element_type=f32) + prib_ref[...]
    mu_pr = ppT[:D_LAT]
    sg_pr = _softplus(ppT[D_LAT:] + SOFTPLUS_BIAS)

    embT = jnp.dot(emb_ref[...], xt, preferred_element_type=f32) + embb_ref[...]
    emb_r = jnp.broadcast_to(embT[:, None, :], (D_EMB, P, BB)).reshape(D_EMB, R)
    y_r = jnp.broadcast_to(yt[:, None, :], (D_DATA, P, BB)).reshape(D_DATA, R)
    inpT = jnp.concatenate([emb_r, zT, y_r], axis=0)
    giT = jnp.dot(wi_ref[...], inpT, preferred_element_type=f32) + bi_ref[...]
    ghT = jnp.dot(wh_ref[...], hT, preferred_element_type=f32) + bh_ref[...]
    r = jax.nn.sigmoid(giT[:D_HID] + ghT[:D_HID])
    zg = jax.nn.sigmoid(giT[D_HID:2 * D_HID] + ghT[D_HID:2 * D_HID])
    n = jnp.tanh(giT[2 * D_HID:] + r * ghT[2 * D_HID:])
    hT = (1.0 - zg) * n + zg * hT
    h_s[...] = hT

    qqT = jnp.dot(enc_ref[...], hT, preferred_element_type=f32) + encb_ref[...]
    mu_po = qqT[:D_LAT]
    sg_po = _softplus(qqT[D_LAT:] + SOFTPLUS_BIAS)

    ek0, ek1, ek2 = _sel_key(t, _EPS_KEYS)
    ek1_off = ek1 + (ib * (BB * P * D_LAT)).astype(jnp.uint32)
    bits_e = _threefry_bits(pose_s[...] + ek1_off, ek0, ek1, ek2)
    u_e = jnp.maximum(f32(NORM_LO),
                      _bits_to_unit_float(bits_e) * f32(NORM_SCALE) + f32(NORM_LO))
    eps = (f32(SQRT2) * _erfinv(u_e)).reshape(D_LAT, R)
    z_new = mu_po + sg_po * eps

    dpT = jnp.dot(dec_ref[...], z_new, preferred_element_type=f32) + decb_ref[...]
    mu_d = dpT[0:1]
    sg_d = _softplus(dpT[1:2] + SOFTPLUS_BIAS)
    rt = y_r[0:1]
    ch = y_r[1:2]
    data_lp = (_normal_logprob(rt, mu_d, sg_d)
               + ch * jax.nn.log_sigmoid(mu_d)
               + (1.0 - ch) * jax.nn.log_sigmoid(-mu_d))

    la_row = (jnp.sum(_normal_logprob(z_new, mu_pr, sg_pr), axis=0, keepdims=True)
              + data_lp
              - jnp.sum(_normal_logprob(z_new, mu_po, sg_po), axis=0, keepdims=True))
    la = la_row.reshape(1, P, BB)

    lp3 = lw_s[...].reshape(1, P, BB) + la
    m = jnp.max(lp3, axis=1, keepdims=True)
    lse = jnp.log(jnp.sum(jnp.exp(lp3 - m), axis=1, keepdims=True)) + m
    acc_s[...] += lse.reshape(1, BB)
    lw3 = lp3 - lse

    m2 = jnp.max(2.0 * lw3, axis=1, keepdims=True)
    lse2 = jnp.log(jnp.sum(jnp.exp(2.0 * lw3 - m2), axis=1, keepdims=True)) + m2
    need = jnp.exp(-lse2) < float(P // 2)

    gk0, gk1, gk2 = _sel_key(t, _GUM_KEYS)
    gk1_off = gk1 + (ib * (BB * P)).astype(jnp.uint32)
    posg = posg_s[...]
    bits_g = _threefry_bits(posg + gk1_off, gk0, gk1, gk2)
    u_g = jnp.maximum(f32(TINY), _bits_to_unit_float(bits_g) + f32(TINY))
    gum3 = -jnp.log(-jnp.log(u_g))

    s3 = gum3 + lw3.reshape(P, 1, BB)
    m3 = jnp.max(s3, axis=0, keepdims=True)
    posg_i = posg.astype(jnp.int32)
    cand = jnp.where(s3 == m3, posg_i, jnp.int32(2 ** 30))
    idx3 = (jnp.min(cand, axis=0, keepdims=True)
            - posg_i[0:1])

    z3 = z_new.reshape(D_LAT, P, BB)
    zres3 = jnp.broadcast_to(z3[:, 0:1, :], (D_LAT, P, BB))
    for k in range(1, P):
        sel = jnp.broadcast_to(idx3 == k, (D_LAT, P, BB))
        zres3 = jnp.where(sel, jnp.broadcast_to(z3[:, k:k + 1, :], (D_LAT, P, BB)),
                          zres3)
    zres = zres3.reshape(D_LAT, R)

    mask = need.astype(f32)
    mask_r = jnp.broadcast_to(mask, (D_LAT, P, BB)).reshape(D_LAT, R)
    z_new = zres * mask_r + z_new * (1.0 - mask_r)
    lw3 = f32(NEG_LOG_P) * mask + lw3 * (1.0 - mask)

    z_s[...] = z_new
    lw_s[...] = lw3.reshape(P, BB)

    @pl.when(t == T - 1)
    def _emit():
        out_ref[...] = acc_s[...]


def kernel(x, y, emb_W, emb_b, gru_Wi, gru_Wh, gru_bi, gru_bh,
           enc_W, enc_b, pri_W, pri_b, dec_W, dec_b):
    f32 = jnp.float32
    xT = x.transpose(1, 2, 0)
    yT = y.transpose(1, 2, 0)

    grid = (NB, T)
    specs = [
        pl.BlockSpec((1, D_DATA, BB), lambda i, t: (t, 0, i)),
        pl.BlockSpec((1, D_DATA, BB), lambda i, t: (t, 0, i)),
        pl.BlockSpec((96, 34), lambda i, t: (0, 0)),
        pl.BlockSpec((96, 32), lambda i, t: (0, 0)),
        pl.BlockSpec((96, 1), lambda i, t: (0, 0)),
        pl.BlockSpec((96, 1), lambda i, t: (0, 0)),
        pl.BlockSpec((32, 32), lambda i, t: (0, 0)),
        pl.BlockSpec((32, 1), lambda i, t: (0, 0)),
        pl.BlockSpec((32, 16), lambda i, t: (0, 0)),
        pl.BlockSpec((32, 1), lambda i, t: (0, 0)),
        pl.BlockSpec((2, 16), lambda i, t: (0, 0)),
        pl.BlockSpec((2, 1), lambda i, t: (0, 0)),
        pl.BlockSpec((16, 2), lambda i, t: (0, 0)),
        pl.BlockSpec((16, 1), lambda i, t: (0, 0)),
    ]
    out = pl.pallas_call(
        _fivo_step,
        grid=grid,
        in_specs=specs,
        out_specs=pl.BlockSpec((1, BB), lambda i, t: (0, i)),
        out_shape=jax.ShapeDtypeStruct((1, B), f32),
        scratch_shapes=[
            pltpu.VMEM((D_LAT, R), f32),
            pltpu.VMEM((D_HID, R), f32),
            pltpu.VMEM((P, BB), f32),
            pltpu.VMEM((1, BB), f32),
            pltpu.VMEM((D_LAT, P, BB), jnp.uint32),
            pltpu.VMEM((P, P, BB), jnp.uint32),
        ],
    )(
        xT, yT,
        gru_Wi, gru_Wh, gru_bi[:, None], gru_bh[:, None],
        enc_W, enc_b[:, None], pri_W, pri_b[:, None],
        dec_W, dec_b[:, None], emb_W, emb_b[:, None],
    )
    return out.reshape(B, 1)


# speedup vs baseline: 1.3131x; 1.3131x over previous
import math

import jax
import jax.numpy as jnp
import numpy as np
from jax.experimental import pallas as pl
from jax.experimental.pallas import tpu as pltpu

B = 1024
T = 20
P = 50
D_DATA = 2
D_EMB = 16
D_LAT = 16
D_HID = 32
SOFTPLUS_BIAS = 0.5413248538970947

BB = 128
NB = B // BB
R = BB * P
HALF_LOG_2PI = 0.5 * math.log(2.0 * math.pi)
NEG_LOG_P = -math.log(float(P))
TINY = float(np.finfo(np.float32).tiny)
NORM_LO = float(np.nextafter(np.float32(-1.0), np.float32(0.0)))
NORM_SCALE = float(np.float32(1.0) - np.float32(NORM_LO))
SQRT2 = float(np.float32(np.sqrt(2.0)))


def _np_threefry2x32(k0, k1, c0, c1):
    def rotl(x, r):
        return np.uint32((int(x) << r | int(x) >> (32 - r)) & 0xFFFFFFFF)
    ks0, ks1 = np.uint32(k0), np.uint32(k1)
    ks2 = np.uint32(ks0 ^ ks1 ^ np.uint32(0x1BD11BDA))
    x0 = np.uint32((int(c0) + int(ks0)) & 0xFFFFFFFF)
    x1 = np.uint32((int(c1) + int(ks1)) & 0xFFFFFFFF)
    rots = [(13, 15, 26, 6), (17, 29, 16, 24)]
    adds = [(ks1, ks2, 1), (ks2, ks0, 2), (ks0, ks1, 3), (ks1, ks2, 4), (ks2, ks0, 5)]
    for i, (a0, a1, inc) in enumerate(adds):
        for r in rots[i % 2]:
            x0 = np.uint32((int(x0) + int(x1)) & 0xFFFFFFFF)
            x1 = rotl(x1, r)
            x1 = np.uint32(x1 ^ x0)
        x0 = np.uint32((int(x0) + int(a0)) & 0xFFFFFFFF)
        x1 = np.uint32((int(x1) + int(a1) + inc) & 0xFFFFFFFF)
    return x0, x1


def _fold_keys():
    eps_keys, gum_keys = [], []
    for t in range(T):
        for lst, d in ((eps_keys, 2 * t), (gum_keys, 2 * t + 1)):
            f0, f1 = _np_threefry2x32(np.uint32(0), np.uint32(42),
                                      np.uint32(0), np.uint32(d))
            lst.append((int(f0), int(f1), int(f0 ^ f1 ^ 0x1BD11BDA)))
    return eps_keys, gum_keys


_EPS_KEYS, _GUM_KEYS = _fold_keys()

_ERFINV_LT = (2.81022636e-08, 3.43273939e-07, -3.5233877e-06, -4.39150654e-06,
              0.00021858087, -0.00125372503, -0.00417768164, 0.246640727,
              1.50140941)
_ERFINV_GE = (-0.000200214257, 0.000100950558, 0.00134934322, -0.00367342844,
              0.00573950773, -0.0076224613, 0.00943887047, 1.00167406,
              2.83297682)


def _sel_key(t, keys):
    outs = []
    for j in range(3):
        acc = jnp.uint32(keys[0][j])
        for i in range(1, T):
            acc = jax.lax.select(t == i, jnp.uint32(keys[i][j]), acc)
        outs.append(acc)
    return outs


def _threefry_bits(pos_plus_k1, k0, k1, ks2):
    u32 = jnp.uint32
    x1 = pos_plus_k1

    def quarter(x0, x1, rots):
        for r in rots:
            x0 = x0 + x1
            x1 = (x1 << u32(r)) | (x1 >> u32(32 - r))
            x1 = x1 ^ x0
        return x0, x1

    ra, rb = (13, 15, 26, 6), (17, 29, 16, 24)
    x0 = x1 + k0
    x1 = (x1 << u32(ra[0])) | (x1 >> u32(32 - ra[0]))
    x1 = x1 ^ x0
    for r in ra[1:]:
        x0 = x0 + x1
        x1 = (x1 << u32(r)) | (x1 >> u32(32 - r))
        x1 = x1 ^ x0
    x0 = x0 + k1
    x1 = x1 + ks2 + u32(1)
    x0, x1 = quarter(x0, x1, rb)
    x0 = x0 + ks2
    x1 = x1 + k0 + u32(2)
    x0, x1 = quarter(x0, x1, ra)
    x0 = x0 + k0
    x1 = x1 + k1 + u32(3)
    x0, x1 = quarter(x0, x1, rb)
    x0 = x0 + k1
    x1 = x1 + ks2 + u32(4)
    x0, x1 = quarter(x0, x1, ra)
    x0 = x0 + ks2
    x1 = x1 + k0 + u32(5)
    return x0 ^ x1


def _bits_to_unit_float(bits):
    fb = (bits >> jnp.uint32(9)) | jnp.uint32(0x3F800000)
    return jax.lax.bitcast_convert_type(fb, jnp.float32) - 1.0


def _erfinv(x):
    w = -jnp.log1p(-x * x)
    wl = w - 2.5
    p1 = jnp.float32(_ERFINV_LT[0])
    for c in _ERFINV_LT[1:]:
        p1 = p1 * wl + jnp.float32(c)
    ws = jnp.sqrt(w) - 3.0
    p2 = jnp.float32(_ERFINV_GE[0])
    for c in _ERFINV_GE[1:]:
        p2 = p2 * ws + jnp.float32(c)
    return jnp.where(w < 5.0, p1, p2) * x


def _softplus(v):
    return jnp.logaddexp(v, 0.0)


def _normal_logprob(x, mu, sig):
    return -0.5 * ((x - mu) / sig) ** 2 - jnp.log(sig) - HALF_LOG_2PI


def _fivo_step(xt_ref, yt_ref,
               wi_ref, wh_ref, bi_ref, bh_ref,
               enc_ref, encb_ref, pri_ref, prib_ref,
               dec_ref, decb_ref, emb_ref, embb_ref,
               out_ref, z_s, h_s, lw_s, acc_s, pose_s, posg_s):
    ib = pl.program_id(0)
    t = pl.program_id(1)
    f32 = jnp.float32

    @pl.when(t == 0)
    def _init():
        z_s[...] = jnp.zeros_like(z_s)
        h_s[...] = jnp.zeros_like(h_s)
        lw_s[...] = jnp.zeros_like(lw_s)
        acc_s[...] = jnp.zeros_like(acc_s)

    @pl.when((t == 0) & (ib == 0))
    def _init_pos():
        i32 = jnp.int32
        d_i = jax.lax.broadcasted_iota(i32, (D_LAT, P, BB), 0)
        p_i = jax.lax.broadcasted_iota(i32, (D_LAT, P, BB), 1)
        b_i = jax.lax.broadcasted_iota(i32, (D_LAT, P, BB), 2)
        pose_s[...] = (b_i * (P * D_LAT) + p_i * D_LAT + d_i).astype(jnp.uint32)
        k_i = jax.lax.broadcasted_iota(i32, (P, P, BB), 0)
        pn_i = jax.lax.broadcasted_iota(i32, (P, P, BB), 1)
        gb_i = jax.lax.broadcasted_iota(i32, (P, P, BB), 2)
        posg_s[...] = (pn_i * (B * P) + gb_i * P + k_i).astype(jnp.uint32)

    xt = xt_ref[0]
    yt = yt_ref[0]
    zT = z_s[...]
    hT = h_s[...]

    ppT = jnp.dot(pri_ref[...], zT, preferred_element_type=f32) + prib_ref[...]
    mu_pr = ppT[:D_LAT]
    sg_pr = _softplus(ppT[D_LAT:] + SOFTPLUS_BIAS)

    embT = jnp.dot(emb_ref[...], xt, preferred_element_type=f32) + embb_ref[...]
    emb_r = jnp.broadcast_to(embT[:, None, :], (D_EMB, P, BB)).reshape(D_EMB, R)
    y_r = jnp.broadcast_to(yt[:, None, :], (D_DATA, P, BB)).reshape(D_DATA, R)
    inpT = jnp.concatenate([emb_r, zT, y_r], axis=0)
    giT = jnp.dot(wi_ref[...], inpT, preferred_element_type=f32) + bi_ref[...]
    ghT = jnp.dot(wh_ref[...], hT, preferred_element_type=f32) + bh_ref[...]
    r = jax.nn.sigmoid(giT[:D_HID] + ghT[:D_HID])
    zg = jax.nn.sigmoid(giT[D_HID:2 * D_HID] + ghT[D_HID:2 * D_HID])
    n = jnp.tanh(giT[2 * D_HID:] + r * ghT[2 * D_HID:])
    hT = (1.0 - zg) * n + zg * hT
    h_s[...] = hT

    qqT = jnp.dot(enc_ref[...], hT, preferred_element_type=f32) + encb_ref[...]
    mu_po = qqT[:D_LAT]
    sg_po = _softplus(qqT[D_LAT:] + SOFTPLUS_BIAS)

    ek0, ek1, ek2 = _sel_key(t, _EPS_KEYS)
    ek1_off = ek1 + (ib * (BB * P * D_LAT)).astype(jnp.uint32)
    bits_e = _threefry_bits(pose_s[...] + ek1_off, ek0, ek1, ek2)
    u_e = jnp.maximum(f32(NORM_LO),
                      _bits_to_unit_float(bits_e) * f32(NORM_SCALE) + f32(NORM_LO))
    eps = (f32(SQRT2) * _erfinv(u_e)).reshape(D_LAT, R)
    z_new = mu_po + sg_po * eps

    dpT = jnp.dot(dec_ref[...], z_new, preferred_element_type=f32) + decb_ref[...]
    mu_d = dpT[0:1]
    sg_d = _softplus(dpT[1:2] + SOFTPLUS_BIAS)
    rt = y_r[0:1]
    ch = y_r[1:2]
    data_lp = (_normal_logprob(rt, mu_d, sg_d)
               + ch * jax.nn.log_sigmoid(mu_d)
               + (1.0 - ch) * jax.nn.log_sigmoid(-mu_d))

    la_row = (jnp.sum(_normal_logprob(z_new, mu_pr, sg_pr), axis=0, keepdims=True)
              + data_lp
              - jnp.sum(_normal_logprob(z_new, mu_po, sg_po), axis=0, keepdims=True))
    la = la_row.reshape(1, P, BB)

    lp3 = lw_s[...].reshape(1, P, BB) + la
    m = jnp.max(lp3, axis=1, keepdims=True)
    lse = jnp.log(jnp.sum(jnp.exp(lp3 - m), axis=1, keepdims=True)) + m
    acc_s[...] += lse.reshape(1, BB)
    lw3 = lp3 - lse

    m2 = jnp.max(2.0 * lw3, axis=1, keepdims=True)
    lse2 = jnp.log(jnp.sum(jnp.exp(2.0 * lw3 - m2), axis=1, keepdims=True)) + m2
    need = jnp.exp(-lse2) < float(P // 2)

    gk0, gk1, gk2 = _sel_key(t, _GUM_KEYS)
    gk1_off = gk1 + (ib * (BB * P)).astype(jnp.uint32)
    posg = posg_s[...]
    bits_g = _threefry_bits(posg + gk1_off, gk0, gk1, gk2)
    u_g = jnp.maximum(f32(TINY), _bits_to_unit_float(bits_g) + f32(TINY))
    gum3 = -jnp.log(-jnp.log(u_g))

    s3 = gum3 + lw3.reshape(P, 1, BB)
    m3 = jnp.max(s3, axis=0, keepdims=True)
    posg_i = posg.astype(jnp.int32)
    cand = jnp.where(s3 == m3, posg_i, jnp.int32(2 ** 30))
    idx3 = (jnp.min(cand, axis=0, keepdims=True)
            - posg_i[0:1])

    z3 = z_new.reshape(D_LAT, P, BB)
    zres3 = jnp.broadcast_to(z3[:, 0:1, :], (D_LAT, P, BB))
    for k in range(1, P):
        sel = jnp.broadcast_to(idx3 == k, (D_LAT, P, BB))
        zres3 = jnp.where(sel, jnp.broadcast_to(z3[:, k:k + 1, :], (D_LAT, P, BB)),
                          zres3)
    zres = zres3.reshape(D_LAT, R)

    mask = need.astype(f32)
    mask_r = jnp.broadcast_to(mask, (D_LAT, P, BB)).reshape(D_LAT, R)
    z_new = zres * mask_r + z_new * (1.0 - mask_r)
    lw3 = f32(NEG_LOG_P) * mask + lw3 * (1.0 - mask)

    z_s[...] = z_new
    lw_s[...] = lw3.reshape(P, BB)

    @pl.when(t == T - 1)
    def _emit():
        out_ref[...] = acc_s[...]


def kernel(x, y, emb_W, emb_b, gru_Wi, gru_Wh, gru_bi, gru_bh,
           enc_W, enc_b, pri_W, pri_b, dec_W, dec_b):
    f32 = jnp.float32
    xT = x.transpose(1, 2, 0)
    yT = y.transpose(1, 2, 0)

    grid = (NB, T)
    specs = [
        pl.BlockSpec((1, D_DATA, BB), lambda i, t: (t, 0, i)),
        pl.BlockSpec((1, D_DATA, BB), lambda i, t: (t, 0, i)),
        pl.BlockSpec((96, 34), lambda i, t: (0, 0)),
        pl.BlockSpec((96, 32), lambda i, t: (0, 0)),
        pl.BlockSpec((96, 1), lambda i, t: (0, 0)),
        pl.BlockSpec((96, 1), lambda i, t: (0, 0)),
        pl.BlockSpec((32, 32), lambda i, t: (0, 0)),
        pl.BlockSpec((32, 1), lambda i, t: (0, 0)),
        pl.BlockSpec((32, 16), lambda i, t: (0, 0)),
        pl.BlockSpec((32, 1), lambda i, t: (0, 0)),
        pl.BlockSpec((2, 16), lambda i, t: (0, 0)),
        pl.BlockSpec((2, 1), lambda i, t: (0, 0)),
        pl.BlockSpec((16, 2), lambda i, t: (0, 0)),
        pl.BlockSpec((16, 1), lambda i, t: (0, 0)),
    ]
    out = pl.pallas_call(
        _fivo_step,
        grid=grid,
        in_specs=specs,
        out_specs=pl.BlockSpec((1, BB), lambda i, t: (0, i)),
        out_shape=jax.ShapeDtypeStruct((1, B), f32),
        scratch_shapes=[
            pltpu.VMEM((D_LAT, R), f32),
            pltpu.VMEM((D_HID, R), f32),
            pltpu.VMEM((P, BB), f32),
            pltpu.VMEM((1, BB), f32),
            pltpu.VMEM((D_LAT, P, BB), jnp.uint32),
            pltpu.VMEM((P, P, BB), jnp.uint32),
        ],
    )(
        xT, yT,
        gru_Wi, gru_Wh, gru_bi[:, None], gru_bh[:, None],
        enc_W, enc_b[:, None], pri_W, pri_b[:, None],
        dec_W, dec_b[:, None], emb_W, emb_b[:, None],
    )
    return out.reshape(B, 1)


# speedup vs baseline: 1.5032x; 1.1448x over previous
import math

import jax
import jax.numpy as jnp
import numpy as np
from jax.experimental import pallas as pl
from jax.experimental.pallas import tpu as pltpu

B = 1024
T = 20
P = 50
D_DATA = 2
D_EMB = 16
D_LAT = 16
D_HID = 32
SOFTPLUS_BIAS = 0.5413248538970947

BB = 128
NB = B // BB
R = BB * P
HALF_LOG_2PI = 0.5 * math.log(2.0 * math.pi)
NEG_LOG_P = -math.log(float(P))
TINY = float(np.finfo(np.float32).tiny)
NORM_LO = float(np.nextafter(np.float32(-1.0), np.float32(0.0)))
NORM_SCALE = float(np.float32(1.0) - np.float32(NORM_LO))
SQRT2 = float(np.float32(np.sqrt(2.0)))


def _np_threefry2x32(k0, k1, c0, c1):
    def rotl(x, r):
        return np.uint32((int(x) << r | int(x) >> (32 - r)) & 0xFFFFFFFF)
    ks0, ks1 = np.uint32(k0), np.uint32(k1)
    ks2 = np.uint32(ks0 ^ ks1 ^ np.uint32(0x1BD11BDA))
    x0 = np.uint32((int(c0) + int(ks0)) & 0xFFFFFFFF)
    x1 = np.uint32((int(c1) + int(ks1)) & 0xFFFFFFFF)
    rots = [(13, 15, 26, 6), (17, 29, 16, 24)]
    adds = [(ks1, ks2, 1), (ks2, ks0, 2), (ks0, ks1, 3), (ks1, ks2, 4), (ks2, ks0, 5)]
    for i, (a0, a1, inc) in enumerate(adds):
        for r in rots[i % 2]:
            x0 = np.uint32((int(x0) + int(x1)) & 0xFFFFFFFF)
            x1 = rotl(x1, r)
            x1 = np.uint32(x1 ^ x0)
        x0 = np.uint32((int(x0) + int(a0)) & 0xFFFFFFFF)
        x1 = np.uint32((int(x1) + int(a1) + inc) & 0xFFFFFFFF)
    return x0, x1


def _fold_keys():
    eps_keys, gum_keys = [], []
    for t in range(T):
        for lst, d in ((eps_keys, 2 * t), (gum_keys, 2 * t + 1)):
            f0, f1 = _np_threefry2x32(np.uint32(0), np.uint32(42),
                                      np.uint32(0), np.uint32(d))
            lst.append((int(f0), int(f1), int(f0 ^ f1 ^ 0x1BD11BDA)))
    return eps_keys, gum_keys


_EPS_KEYS, _GUM_KEYS = _fold_keys()

_ERFINV_LT = (2.81022636e-08, 3.43273939e-07, -3.5233877e-06, -4.39150654e-06,
              0.00021858087, -0.00125372503, -0.00417768164, 0.246640727,
              1.50140941)
_ERFINV_GE = (-0.000200214257, 0.000100950558, 0.00134934322, -0.00367342844,
              0.00573950773, -0.0076224613, 0.00943887047, 1.00167406,
              2.83297682)


def _sel_key(t, keys):
    outs = []
    for j in range(3):
        acc = jnp.uint32(keys[0][j])
        for i in range(1, T):
            acc = jax.lax.select(t == i, jnp.uint32(keys[i][j]), acc)
        outs.append(acc)
    return outs


def _threefry_bits(pos_plus_k1, k0, k1, ks2):
    u32 = jnp.uint32
    x1 = pos_plus_k1

    def quarter(x0, x1, rots):
        for r in rots:
            x0 = x0 + x1
            x1 = (x1 << u32(r)) | (x1 >> u32(32 - r))
            x1 = x1 ^ x0
        return x0, x1

    ra, rb = (13, 15, 26, 6), (17, 29, 16, 24)
    x0 = x1 + k0
    x1 = (x1 << u32(ra[0])) | (x1 >> u32(32 - ra[0]))
    x1 = x1 ^ x0
    for r in ra[1:]:
        x0 = x0 + x1
        x1 = (x1 << u32(r)) | (x1 >> u32(32 - r))
        x1 = x1 ^ x0
    x0 = x0 + k1
    x1 = x1 + ks2 + u32(1)
    x0, x1 = quarter(x0, x1, rb)
    x0 = x0 + ks2
    x1 = x1 + k0 + u32(2)
    x0, x1 = quarter(x0, x1, ra)
    x0 = x0 + k0
    x1 = x1 + k1 + u32(3)
    x0, x1 = quarter(x0, x1, rb)
    x0 = x0 + k1
    x1 = x1 + ks2 + u32(4)
    x0, x1 = quarter(x0, x1, ra)
    x0 = x0 + ks2
    x1 = x1 + k0 + u32(5)
    return x0 ^ x1


def _bits_to_unit_float(bits):
    fb = (bits >> jnp.uint32(9)) | jnp.uint32(0x3F800000)
    return jax.lax.bitcast_convert_type(fb, jnp.float32) - 1.0


def _erfinv(x):
    w = -jnp.log1p(-x * x)
    wl = w - 2.5
    p1 = jnp.float32(_ERFINV_LT[0])
    for c in _ERFINV_LT[1:]:
        p1 = p1 * wl + jnp.float32(c)
    ws = jnp.sqrt(w) - 3.0
    p2 = jnp.float32(_ERFINV_GE[0])
    for c in _ERFINV_GE[1:]:
        p2 = p2 * ws + jnp.float32(c)
    return jnp.where(w < 5.0, p1, p2) * x


def _softplus(v):
    return jnp.logaddexp(v, 0.0)


def _normal_logprob(x, mu, sig):
    return -0.5 * ((x - mu) / sig) ** 2 - jnp.log(sig) - HALF_LOG_2PI


def _fivo_step(xt_ref, yt_ref,
               wi_ref, wh_ref, bi_ref, bh_ref,
               enc_ref, encb_ref, pri_ref, prib_ref,
               dec_ref, decb_ref, emb_ref, embb_ref,
               out_ref, z_s, h_s, lw_s, acc_s, pose_s, posg_s):
    ib = pl.program_id(0)
    t = pl.program_id(1)
    f32 = jnp.float32

    @pl.when(t == 0)
    def _init():
        z_s[...] = jnp.zeros_like(z_s)
        h_s[...] = jnp.zeros_like(h_s)
        lw_s[...] = jnp.zeros_like(lw_s)
        acc_s[...] = jnp.zeros_like(acc_s)

    @pl.when((t == 0) & (ib == 0))
    def _init_pos():
        i32 = jnp.int32
        d_i = jax.lax.broadcasted_iota(i32, (D_LAT, P, BB), 0)
        p_i = jax.lax.broadcasted_iota(i32, (D_LAT, P, BB), 1)
        b_i = jax.lax.broadcasted_iota(i32, (D_LAT, P, BB), 2)
        pose_s[...] = (b_i * (P * D_LAT) + p_i * D_LAT + d_i).astype(jnp.uint32)
        k_i = jax.lax.broadcasted_iota(i32, (P, P, BB), 0)
        pn_i = jax.lax.broadcasted_iota(i32, (P, P, BB), 1)
        gb_i = jax.lax.broadcasted_iota(i32, (P, P, BB), 2)
        posg_s[...] = (pn_i * (B * P) + gb_i * P + k_i).astype(jnp.uint32)

    xt = xt_ref[0]
    yt = yt_ref[0]
    zT = z_s[...]
    hT = h_s[...]

    ppT = jnp.dot(pri_ref[...], zT, preferred_element_type=f32) + prib_ref[...]
    mu_pr = ppT[:D_LAT]
    sg_pr = _softplus(ppT[D_LAT:] + SOFTPLUS_BIAS)

    embT = jnp.dot(emb_ref[...], xt, preferred_element_type=f32) + embb_ref[...]
    emb_r = jnp.broadcast_to(embT[:, None, :], (D_EMB, P, BB)).reshape(D_EMB, R)
    y_r = jnp.broadcast_to(yt[:, None, :], (D_DATA, P, BB)).reshape(D_DATA, R)
    inpT = jnp.concatenate([emb_r, zT, y_r], axis=0)
    giT = jnp.dot(wi_ref[...], inpT, preferred_element_type=f32) + bi_ref[...]
    ghT = jnp.dot(wh_ref[...], hT, preferred_element_type=f32) + bh_ref[...]
    r = jax.nn.sigmoid(giT[:D_HID] + ghT[:D_HID])
    zg = jax.nn.sigmoid(giT[D_HID:2 * D_HID] + ghT[D_HID:2 * D_HID])
    n = jnp.tanh(giT[2 * D_HID:] + r * ghT[2 * D_HID:])
    hT = (1.0 - zg) * n + zg * hT
    h_s[...] = hT

    qqT = jnp.dot(enc_ref[...], hT, preferred_element_type=f32) + encb_ref[...]
    mu_po = qqT[:D_LAT]
    sg_po = _softplus(qqT[D_LAT:] + SOFTPLUS_BIAS)

    ek0, ek1, ek2 = _sel_key(t, _EPS_KEYS)
    ek1_off = ek1 + (ib * (BB * P * D_LAT)).astype(jnp.uint32)
    bits_e = _threefry_bits(pose_s[...] + ek1_off, ek0, ek1, ek2)
    u_e = jnp.maximum(f32(NORM_LO),
                      _bits_to_unit_float(bits_e) * f32(NORM_SCALE) + f32(NORM_LO))
    eps = (f32(SQRT2) * _erfinv(u_e)).reshape(D_LAT, R)
    z_new = mu_po + sg_po * eps

    dpT = jnp.dot(dec_ref[...], z_new, preferred_element_type=f32) + decb_ref[...]
    mu_d = dpT[0:1]
    sg_d = _softplus(dpT[1:2] + SOFTPLUS_BIAS)
    rt = y_r[0:1]
    ch = y_r[1:2]
    data_lp = (_normal_logprob(rt, mu_d, sg_d)
               + ch * jax.nn.log_sigmoid(mu_d)
               + (1.0 - ch) * jax.nn.log_sigmoid(-mu_d))

    la_row = (jnp.sum(_normal_logprob(z_new, mu_pr, sg_pr), axis=0, keepdims=True)
              + data_lp
              - jnp.sum(_normal_logprob(z_new, mu_po, sg_po), axis=0, keepdims=True))
    la = la_row.reshape(1, P, BB)

    lp3 = lw_s[...].reshape(1, P, BB) + la
    m = jnp.max(lp3, axis=1, keepdims=True)
    lse = jnp.log(jnp.sum(jnp.exp(lp3 - m), axis=1, keepdims=True)) + m
    acc_s[...] += lse.reshape(1, BB)
    lw3 = lp3 - lse

    m2 = jnp.max(2.0 * lw3, axis=1, keepdims=True)
    lse2 = jnp.log(jnp.sum(jnp.exp(2.0 * lw3 - m2), axis=1, keepdims=True)) + m2
    need = jnp.exp(-lse2) < float(P // 2)

    gk0, gk1, gk2 = _sel_key(t, _GUM_KEYS)
    gk1_off = gk1 + (ib * (BB * P)).astype(jnp.uint32)
    posg = posg_s[...]
    bits_g = _threefry_bits(posg + gk1_off, gk0, gk1, gk2)
    u_g = jnp.maximum(f32(TINY), _bits_to_unit_float(bits_g) + f32(TINY))
    gum3 = -jnp.log(-jnp.log(u_g))

    s3 = gum3 + lw3.reshape(P, 1, BB)
    m3 = jnp.max(s3, axis=0, keepdims=True)
    posg_i = posg.astype(jnp.int32)
    cand = jnp.where(s3 == m3, posg_i, jnp.int32(2 ** 30))
    idx3 = (jnp.min(cand, axis=0, keepdims=True)
            - posg_i[0:1])

    z3 = z_new.reshape(D_LAT, P, BB)
    zbk = jnp.transpose(z3, (0, 2, 1))
    idxb = jnp.transpose(idx3, (0, 2, 1))
    idxb_r = jnp.broadcast_to(idxb, (D_LAT, BB, P))
    zres_bk = jnp.take_along_axis(zbk, idxb_r, axis=2)
    zres = jnp.transpose(zres_bk, (0, 2, 1)).reshape(D_LAT, R)

    mask = need.astype(f32)
    mask_r = jnp.broadcast_to(mask, (D_LAT, P, BB)).reshape(D_LAT, R)
    z_new = zres * mask_r + z_new * (1.0 - mask_r)
    lw3 = f32(NEG_LOG_P) * mask + lw3 * (1.0 - mask)

    z_s[...] = z_new
    lw_s[...] = lw3.reshape(P, BB)

    @pl.when(t == T - 1)
    def _emit():
        out_ref[...] = acc_s[...]


def kernel(x, y, emb_W, emb_b, gru_Wi, gru_Wh, gru_bi, gru_bh,
           enc_W, enc_b, pri_W, pri_b, dec_W, dec_b):
    f32 = jnp.float32
    xT = x.transpose(1, 2, 0)
    yT = y.transpose(1, 2, 0)

    grid = (NB, T)
    specs = [
        pl.BlockSpec((1, D_DATA, BB), lambda i, t: (t, 0, i)),
        pl.BlockSpec((1, D_DATA, BB), lambda i, t: (t, 0, i)),
        pl.BlockSpec((96, 34), lambda i, t: (0, 0)),
        pl.BlockSpec((96, 32), lambda i, t: (0, 0)),
        pl.BlockSpec((96, 1), lambda i, t: (0, 0)),
        pl.BlockSpec((96, 1), lambda i, t: (0, 0)),
        pl.BlockSpec((32, 32), lambda i, t: (0, 0)),
        pl.BlockSpec((32, 1), lambda i, t: (0, 0)),
        pl.BlockSpec((32, 16), lambda i, t: (0, 0)),
        pl.BlockSpec((32, 1), lambda i, t: (0, 0)),
        pl.BlockSpec((2, 16), lambda i, t: (0, 0)),
        pl.BlockSpec((2, 1), lambda i, t: (0, 0)),
        pl.BlockSpec((16, 2), lambda i, t: (0, 0)),
        pl.BlockSpec((16, 1), lambda i, t: (0, 0)),
    ]
    out = pl.pallas_call(
        _fivo_step,
        grid=grid,
        in_specs=specs,
        out_specs=pl.BlockSpec((1, BB), lambda i, t: (0, i)),
        out_shape=jax.ShapeDtypeStruct((1, B), f32),
        scratch_shapes=[
            pltpu.VMEM((D_LAT, R), f32),
            pltpu.VMEM((D_HID, R), f32),
            pltpu.VMEM((P, BB), f32),
            pltpu.VMEM((1, BB), f32),
            pltpu.VMEM((D_LAT, P, BB), jnp.uint32),
            pltpu.VMEM((P, P, BB), jnp.uint32),
        ],
    )(
        xT, yT,
        gru_Wi, gru_Wh, gru_bi[:, None], gru_bh[:, None],
        enc_W, enc_b[:, None], pri_W, pri_b[:, None],
        dec_W, dec_b[:, None], emb_W, emb_b[:, None],
    )
    return out.reshape(B, 1)
